# Initial kernel scaffold; baseline (speedup 1.0000x reference)
import jax
import jax.numpy as jnp
from jax.experimental import pallas as pl


def kernel(features, dir_table, ext_table, type_table, W, b):
    raise NotImplementedError("write your pallas kernel here")


# speedup vs baseline: 3.8961x; 3.8961x over previous
import functools

import jax
import jax.numpy as jnp
from jax import lax
from jax.experimental import pallas as pl
from jax.experimental.pallas import tpu as pltpu
from jax.experimental.pallas import tpu_sc as plsc

B = 16384
OUT_DIM = 128
LUT = 512
NC, NS = 2, 16
NW = NC * NS
BPW = B // NW
IG = 4


def _table_kernel(dir_ref, ext_ref, typ_ref, w_ref, b_ref, lut_ref):
    w = w_ref[...]
    pd = jnp.dot(dir_ref[...], w[0:5, :], preferred_element_type=jnp.float32)
    pe = jnp.dot(ext_ref[...], w[5:10, :], preferred_element_type=jnp.float32)
    pt = jnp.dot(typ_ref[...], w[10:15, :], preferred_element_type=jnp.float32)
    pet = (pe[:, None, :] + pt[None, :, :]).reshape(64, OUT_DIM)
    full = (pd[:, None, :] + pet[None, :, :]).reshape(LUT, OUT_DIM)
    lut_ref[...] = jax.nn.sigmoid(full + b_ref[...][None, :])


def _build_table(dir8, ext8, typ8, w, b):
    return pl.pallas_call(
        _table_kernel,
        out_shape=jax.ShapeDtypeStruct((LUT, OUT_DIM), jnp.float32),
    )(dir8, ext8, typ8, w, b)


def _sc_gather_body(feat_hbm, lut_hbm, out_hbm, feat_v, cidx, rows_v, sem):
    wid = lax.axis_index("s") * NC + lax.axis_index("c")
    base = wid * BPW
    pltpu.sync_copy(feat_hbm.at[pl.ds(base * 3, BPW * 3)], feat_v)

    lane3 = lax.iota(jnp.int32, 16) * 3
    for j in range(BPW // 16):
        pos = lane3 + (j * 48)
        f0 = plsc.load_gather(feat_v, [pos])
        f1 = plsc.load_gather(feat_v, [pos + 1])
        f2 = plsc.load_gather(feat_v, [pos + 2])
        cidx[j // 8, pl.ds((j % 8) * 16, 16)] = f0 * 64 + f1 * 8 + f2

    copies = [
        pltpu.async_copy(
            lut_hbm.at[cidx.at[g]], rows_v.at[pl.ds(g * 128, 128)], sem
        )
        for g in range(IG)
    ]
    for cp in copies:
        cp.wait()
    pltpu.sync_copy(rows_v, out_hbm.at[pl.ds(base, BPW)])


@functools.cache
def _sc_gather():
    return pl.kernel(
        _sc_gather_body,
        mesh=plsc.VectorSubcoreMesh(
            core_axis_name="c",
            subcore_axis_name="s",
            num_cores=NC,
            num_subcores=NS,
        ),
        out_type=jax.ShapeDtypeStruct((B, OUT_DIM), jnp.float32),
        scratch_types=[
            pltpu.VMEM((BPW * 3,), jnp.int32),
            pltpu.VMEM((IG, 128), jnp.int32),
            pltpu.VMEM((BPW, OUT_DIM), jnp.float32),
            pltpu.SemaphoreType.DMA,
        ],
        compiler_params=pltpu.CompilerParams(needs_layout_passes=False),
    )


def kernel(features, dir_table, ext_table, type_table, W, b):
    f = features.astype(jnp.int32).reshape(-1)
    dir8 = dir_table[:8]
    ext8 = jnp.concatenate([ext_table, jnp.zeros((1, 5), ext_table.dtype)], 0)
    lut = _build_table(dir8, ext8, type_table, W, b)
    return _sc_gather()(f, lut)


# speedup vs baseline: 5.0412x; 1.2939x over previous
import functools

import jax
import jax.numpy as jnp
from jax import lax
from jax.experimental import pallas as pl
from jax.experimental.pallas import tpu as pltpu
from jax.experimental.pallas import tpu_sc as plsc

B = 16384
OUT_DIM = 128
LUT = 512
NC, NS = 2, 16
NW = NC * NS
BPW = B // NW
IG = 4


def _table_kernel(dir_ref, ext_ref, typ_ref, w_ref, b_ref, lut_ref):
    w = w_ref[...]
    pd = jnp.dot(dir_ref[...], w[0:5, :], preferred_element_type=jnp.float32)
    pe = jnp.dot(ext_ref[...], w[5:10, :], preferred_element_type=jnp.float32)
    pt = jnp.dot(typ_ref[...], w[10:15, :], preferred_element_type=jnp.float32)
    pet = (pe[:, None, :] + pt[None, :, :]).reshape(64, OUT_DIM)
    full = (pd[:, None, :] + pet[None, :, :]).reshape(LUT, OUT_DIM)
    lut_ref[...] = jax.nn.sigmoid(full + b_ref[...][None, :])


def _build_table(dir8, ext8, typ8, w, b):
    return pl.pallas_call(
        _table_kernel,
        out_shape=jax.ShapeDtypeStruct((LUT, OUT_DIM), jnp.float32),
    )(dir8, ext8, typ8, w, b)


def _sc_gather_body(cidx_hbm, lut_hbm, out_hbm, cidx, rows_v, *sems):
    gsems, wsem = sems[:IG], sems[IG]
    wid = lax.axis_index("s") * NC + lax.axis_index("c")
    base = wid * BPW
    pltpu.sync_copy(cidx_hbm.at[pl.ds(wid * IG, IG)], cidx)
    gcp = [
        pltpu.async_copy(lut_hbm.at[cidx.at[g]], rows_v.at[g], gsems[g])
        for g in range(IG)
    ]
    wcp = []
    for g in range(IG):
        gcp[g].wait()
        wcp.append(
            pltpu.async_copy(
                rows_v.at[g], out_hbm.at[pl.ds(base + g * 128, 128)], wsem
            )
        )
    for cp in wcp:
        cp.wait()


@functools.cache
def _sc_gather():
    return pl.kernel(
        _sc_gather_body,
        mesh=plsc.VectorSubcoreMesh(
            core_axis_name="c",
            subcore_axis_name="s",
            num_cores=NC,
            num_subcores=NS,
        ),
        out_type=jax.ShapeDtypeStruct((B, OUT_DIM), jnp.float32),
        scratch_types=[
            pltpu.VMEM((IG, 128), jnp.int32),
            pltpu.VMEM((IG, 128, OUT_DIM), jnp.float32),
        ]
        + [pltpu.SemaphoreType.DMA] * (IG + 1),
        compiler_params=pltpu.CompilerParams(needs_layout_passes=False),
    )


def kernel(features, dir_table, ext_table, type_table, W, b):
    f = features.astype(jnp.int32)
    c = (f * jnp.array([64, 8, 1], jnp.int32)).sum(axis=1).reshape(128, 128)
    dir8 = dir_table[:8]
    ext8 = jnp.concatenate([ext_table, jnp.zeros((1, 5), ext_table.dtype)], 0)
    lut = _build_table(dir8, ext8, type_table, W, b)
    return _sc_gather()(c, lut)


# speedup vs baseline: 5.2307x; 1.0376x over previous
import functools

import jax
import jax.numpy as jnp
from jax import lax
from jax.experimental import pallas as pl
from jax.experimental.pallas import tpu as pltpu
from jax.experimental.pallas import tpu_sc as plsc

B = 16384
OUT_DIM = 128
LUT = 512
NC, NS = 2, 16
NW = NC * NS
BPW = B // NW
IG = 8
CH = BPW // IG


def _table_kernel(dir_ref, ext_ref, typ_ref, w_ref, b_ref, lut_ref):
    w = w_ref[...]
    pd = jnp.dot(dir_ref[0:8], w[0:5, :], preferred_element_type=jnp.float32)
    pe7 = jnp.dot(ext_ref[...], w[5:10, :], preferred_element_type=jnp.float32)
    pe = jnp.concatenate([pe7, jnp.zeros((1, OUT_DIM), jnp.float32)], 0)
    pt = jnp.dot(typ_ref[...], w[10:15, :], preferred_element_type=jnp.float32)
    pet = (pe[:, None, :] + pt[None, :, :]).reshape(64, OUT_DIM)
    full = (pd[:, None, :] + pet[None, :, :]).reshape(LUT, OUT_DIM)
    lut_ref[...] = jax.nn.sigmoid(full + b_ref[...][None, :])


def _build_table(dir_t, ext_t, typ_t, w, b):
    return pl.pallas_call(
        _table_kernel,
        out_shape=jax.ShapeDtypeStruct((LUT, OUT_DIM), jnp.float32),
    )(dir_t, ext_t, typ_t, w, b)


def _sc_gather_body(cidx_hbm, lut_hbm, out_hbm, cidx, rows_v, *sems):
    gsems, wsem = sems[:IG], sems[IG]
    wid = lax.axis_index("s") * NC + lax.axis_index("c")
    base = wid * BPW
    pltpu.sync_copy(cidx_hbm.at[pl.ds(wid * IG, IG)], cidx)
    gcp = [
        pltpu.async_copy(lut_hbm.at[cidx.at[g]], rows_v.at[g], gsems[g])
        for g in range(IG)
    ]
    wcp = []
    for g in range(IG):
        gcp[g].wait()
        wcp.append(
            pltpu.async_copy(
                rows_v.at[g], out_hbm.at[pl.ds(base + g * CH, CH)], wsem
            )
        )
    for cp in wcp:
        cp.wait()


@functools.cache
def _sc_gather():
    return pl.kernel(
        _sc_gather_body,
        mesh=plsc.VectorSubcoreMesh(
            core_axis_name="c",
            subcore_axis_name="s",
            num_cores=NC,
            num_subcores=NS,
        ),
        out_type=jax.ShapeDtypeStruct((B, OUT_DIM), jnp.float32),
        scratch_types=[
            pltpu.VMEM((IG, CH), jnp.int32),
            pltpu.VMEM((IG, CH, OUT_DIM), jnp.float32),
        ]
        + [pltpu.SemaphoreType.DMA] * (IG + 1),
        compiler_params=pltpu.CompilerParams(needs_layout_passes=False),
    )


def kernel(features, dir_table, ext_table, type_table, W, b):
    f = features.astype(jnp.int32)
    c = (f * jnp.array([64, 8, 1], jnp.int32)).sum(axis=1).reshape(NW * IG, CH)
    lut = _build_table(dir_table, ext_table, type_table, W, b)
    return _sc_gather()(c, lut)


# speedup vs baseline: 6.7145x; 1.2837x over previous
import functools

import jax
import jax.numpy as jnp
from jax import lax
from jax.experimental import pallas as pl
from jax.experimental.pallas import tpu as pltpu
from jax.experimental.pallas import tpu_sc as plsc

B = 16384
OUT_DIM = 128
LUT = 512
NC, NS = 2, 16
NW = NC * NS
BPW = B // NW
IG = 8
CH = BPW // IG


def _table_kernel(dir_ref, ext_ref, typ_ref, w_ref, b_ref, lut_ref):
    w = w_ref[...]
    pd = jnp.dot(dir_ref[0:8], w[0:5, :], preferred_element_type=jnp.float32)
    pe7 = jnp.dot(ext_ref[...], w[5:10, :], preferred_element_type=jnp.float32)
    pe = jnp.concatenate([pe7, jnp.zeros((1, OUT_DIM), jnp.float32)], 0)
    pt = jnp.dot(typ_ref[...], w[10:15, :], preferred_element_type=jnp.float32)
    pet = (pe[:, None, :] + pt[None, :, :]).reshape(64, OUT_DIM)
    full = (pd[:, None, :] + pet[None, :, :]).reshape(LUT, OUT_DIM)
    lut_ref[...] = jax.nn.sigmoid(full + b_ref[...][None, :])


def _build_table(dir_t, ext_t, typ_t, w, b):
    return pl.pallas_call(
        _table_kernel,
        out_shape=jax.ShapeDtypeStruct((LUT, OUT_DIM), jnp.float32),
    )(dir_t, ext_t, typ_t, w, b)


def _sc_gather_body(cidx_hbm, lut_hbm, out_hbm, cidx, rows_v, lut_sp, *sems):
    gsems, wsem = sems[:IG], sems[IG]
    sid = lax.axis_index("s")
    wid = sid * NC + lax.axis_index("c")
    base = wid * BPW

    @pl.when(sid == 0)
    def _stage_lut():
        pltpu.sync_copy(lut_hbm, lut_sp)

    pltpu.sync_copy(cidx_hbm.at[pl.ds(wid * IG, IG)], cidx)
    plsc.subcore_barrier()
    gcp = [
        pltpu.async_copy(lut_sp.at[cidx.at[g]], rows_v.at[g], gsems[g])
        for g in range(IG)
    ]
    wcp = []
    for g in range(IG):
        gcp[g].wait()
        wcp.append(
            pltpu.async_copy(
                rows_v.at[g], out_hbm.at[pl.ds(base + g * CH, CH)], wsem
            )
        )
    for cp in wcp:
        cp.wait()


@functools.cache
def _sc_gather():
    return pl.kernel(
        _sc_gather_body,
        mesh=plsc.VectorSubcoreMesh(
            core_axis_name="c",
            subcore_axis_name="s",
            num_cores=NC,
            num_subcores=NS,
        ),
        out_type=jax.ShapeDtypeStruct((B, OUT_DIM), jnp.float32),
        scratch_types=[
            pltpu.VMEM((IG, CH), jnp.int32),
            pltpu.VMEM((IG, CH, OUT_DIM), jnp.float32),
            pltpu.VMEM_SHARED((LUT, OUT_DIM), jnp.float32),
        ]
        + [pltpu.SemaphoreType.DMA] * (IG + 1),
        compiler_params=pltpu.CompilerParams(needs_layout_passes=False),
    )


def kernel(features, dir_table, ext_table, type_table, W, b):
    f = features.astype(jnp.int32)
    c = (f * jnp.array([64, 8, 1], jnp.int32)).sum(axis=1).reshape(NW * IG, CH)
    lut = _build_table(dir_table, ext_table, type_table, W, b)
    return _sc_gather()(c, lut)
